# baseline (speedup 1.0000x reference)
import jax
import jax.numpy as jnp
from jax import lax
from jax.experimental import pallas as pl
from jax.experimental.pallas import tpu as pltpu
from jax.experimental.pallas import tpu_sc as plsc

N_VOCAB = 100000
N_EMBED = 64
BATCH = 16384
N_SAMPLES = 5

NC = 2
NS = 16
NW = NC * NS
DIMS_PER_W = N_EMBED // NW

TOTAL = BATCH * (2 + N_SAMPLES)
N_BC = BATCH * (1 + N_SAMPLES)

IC = 4096
UNROLL = 8


def _gather_body(in_idx_hbm, bc_idx_hbm, in_tab_t, out_tab_t, out_t,
                 row_v, idx_vs, out_vs, isems, wsems):
    wid = lax.axis_index("s") * NC + lax.axis_index("c")

    def gather_chunk(idx_v, out_v):
        @plsc.parallel_loop(0, IC, 16, unroll=UNROLL)
        def body(i):
            iv = idx_v[pl.ds(i, 16)]
            iv = lax.iota(jnp.int32, 16) * 64 + (iv & 0)
            out_v[pl.ds(i, 16)] = plsc.load_gather(row_v, [iv])

    passes = []
    for t in range(DIMS_PER_W):
        j = wid * DIMS_PER_W + t
        passes.append((j, in_tab_t, in_idx_hbm, BATCH, 0))
        passes.append((j, out_tab_t, bc_idx_hbm, N_BC, BATCH))

    def first_idx_copy(p):
        _, _, idx_hbm, _, _ = passes[p]
        return pltpu.async_copy(idx_hbm.at[pl.ds(0, IC)], idx_vs[0],
                                isems[0])

    nxt_first = first_idx_copy(0)
    for p, (j, tab, idx_hbm, idx_n, out_off) in enumerate(passes):
        pltpu.sync_copy(tab.at[j], row_v)
        nch = idx_n // IC
        ids = [nxt_first, None]
        wds = [None] * nch
        nxt_first = None
        for c in range(nch):
            b = c % 2
            if c + 1 < nch:
                ids[(c + 1) % 2] = pltpu.async_copy(
                    idx_hbm.at[pl.ds((c + 1) * IC, IC)],
                    idx_vs[(c + 1) % 2], isems[(c + 1) % 2])
            ids[b].wait()
            if c - 2 >= 0:
                wds[c - 2].wait()
            gather_chunk(idx_vs[b], out_vs[b])
            if c == nch - 1 and p + 1 < len(passes):
                nxt_first = first_idx_copy(p + 1)
            wds[c] = pltpu.async_copy(
                out_vs[b], out_t.at[j, pl.ds(out_off + c * IC, IC)],
                wsems[b])
        for c in range(max(0, nch - 2), nch):
            wds[c].wait()


def kernel(input_words, output_words, noise_words, in_embed_weight,
           out_embed_weight):
    bc_idx = jnp.concatenate(
        [output_words.astype(jnp.int32),
         noise_words.reshape(-1).astype(jnp.int32)], axis=0)
    mesh = plsc.VectorSubcoreMesh(core_axis_name="c", subcore_axis_name="s")
    f = pl.kernel(
        _gather_body,
        mesh=mesh,
        out_type=jax.ShapeDtypeStruct((N_EMBED, TOTAL), jnp.float32),
        scratch_types=[
            pltpu.VMEM((N_VOCAB,), jnp.float32),
            [pltpu.VMEM((IC,), jnp.int32)] * 2,
            [pltpu.VMEM((IC,), jnp.float32)] * 2,
            [pltpu.SemaphoreType.DMA] * 2,
            [pltpu.SemaphoreType.DMA] * 2,
        ],
        compiler_params=pltpu.CompilerParams(use_tc_tiling_on_sc=True,
                                             needs_layout_passes=False),
    )
    out_t = f(
        input_words.astype(jnp.int32),
        bc_idx,
        in_embed_weight.T,
        out_embed_weight.T,
    )
    return out_t.T


# speedup vs baseline: 1.0004x; 1.0004x over previous
import jax
import jax.numpy as jnp
from jax import lax
from jax.experimental import pallas as pl
from jax.experimental.pallas import tpu as pltpu
from jax.experimental.pallas import tpu_sc as plsc

N_VOCAB = 100000
N_EMBED = 64
BATCH = 16384
N_SAMPLES = 5

NC = 2
NS = 16
NW = NC * NS
DIMS_PER_W = N_EMBED // NW

TOTAL = BATCH * (2 + N_SAMPLES)
N_BC = BATCH * (1 + N_SAMPLES)

IC = 4096
UNROLL = 2
GANG = 4


def _gather_body(in_idx_hbm, bc_idx_hbm, in_tab_t, out_tab_t, out_t,
                 row_v, idx_vs, out_vs, isems, wsems):
    wid = lax.axis_index("s") * NC + lax.axis_index("c")

    def gather_chunk(idx_v, out_v):
        @plsc.parallel_loop(0, IC, 16 * GANG, unroll=UNROLL)
        def body(i):
            ivs = [idx_v[pl.ds(i + 16 * u, 16)] for u in range(GANG)]
            vals = [plsc.load_gather(row_v, [iv]) for iv in ivs]
            for u in range(GANG):
                out_v[pl.ds(i + 16 * u, 16)] = vals[u]

    passes = []
    for t in range(DIMS_PER_W):
        j = wid * DIMS_PER_W + t
        passes.append((j, in_tab_t, in_idx_hbm, BATCH, 0))
        passes.append((j, out_tab_t, bc_idx_hbm, N_BC, BATCH))

    def first_idx_copy(p):
        _, _, idx_hbm, _, _ = passes[p]
        return pltpu.async_copy(idx_hbm.at[pl.ds(0, IC)], idx_vs[0],
                                isems[0])

    nxt_first = first_idx_copy(0)
    for p, (j, tab, idx_hbm, idx_n, out_off) in enumerate(passes):
        pltpu.sync_copy(tab.at[j], row_v)
        nch = idx_n // IC
        ids = [nxt_first, None]
        wds = [None] * nch
        nxt_first = None
        for c in range(nch):
            b = c % 2
            if c + 1 < nch:
                ids[(c + 1) % 2] = pltpu.async_copy(
                    idx_hbm.at[pl.ds((c + 1) * IC, IC)],
                    idx_vs[(c + 1) % 2], isems[(c + 1) % 2])
            ids[b].wait()
            if c - 2 >= 0:
                wds[c - 2].wait()
            gather_chunk(idx_vs[b], out_vs[b])
            if c == nch - 1 and p + 1 < len(passes):
                nxt_first = first_idx_copy(p + 1)
            wds[c] = pltpu.async_copy(
                out_vs[b], out_t.at[j, pl.ds(out_off + c * IC, IC)],
                wsems[b])
        for c in range(max(0, nch - 2), nch):
            wds[c].wait()


def kernel(input_words, output_words, noise_words, in_embed_weight,
           out_embed_weight):
    bc_idx = jnp.concatenate(
        [output_words.astype(jnp.int32),
         noise_words.reshape(-1).astype(jnp.int32)], axis=0)
    mesh = plsc.VectorSubcoreMesh(core_axis_name="c", subcore_axis_name="s")
    f = pl.kernel(
        _gather_body,
        mesh=mesh,
        out_type=jax.ShapeDtypeStruct((N_EMBED, TOTAL), jnp.float32),
        scratch_types=[
            pltpu.VMEM((N_VOCAB,), jnp.float32),
            [pltpu.VMEM((IC,), jnp.int32)] * 2,
            [pltpu.VMEM((IC,), jnp.float32)] * 2,
            [pltpu.SemaphoreType.DMA] * 2,
            [pltpu.SemaphoreType.DMA] * 2,
        ],
        compiler_params=pltpu.CompilerParams(use_tc_tiling_on_sc=True,
                                             needs_layout_passes=False),
    )
    out_t = f(
        input_words.astype(jnp.int32),
        bc_idx,
        in_embed_weight.T,
        out_embed_weight.T,
    )
    return out_t.T


# speedup vs baseline: 1.0582x; 1.0578x over previous
import jax
import jax.numpy as jnp
from jax import lax
from jax.experimental import pallas as pl
from jax.experimental.pallas import tpu as pltpu
from jax.experimental.pallas import tpu_sc as plsc

N_VOCAB = 100000
N_EMBED = 64
BATCH = 16384
N_SAMPLES = 5

NC = 2
NS = 16
NW = NC * NS
DIMS_PER_W = N_EMBED // NW

TOTAL = BATCH * (2 + N_SAMPLES)
N_BC = BATCH * (1 + N_SAMPLES)

IC = 6144
CH_A = (6144, 6144, 4096)
CH_BC = (6144,) * 16
UNROLL = 2
GANG = 4


def _gather_body(in_idx_hbm, bc_idx_hbm, in_tab_t, out_tab_t, out_t,
                 row_v, idx_vs, out_vs, isems, wsems):
    wid = lax.axis_index("s") * NC + lax.axis_index("c")

    def gather_chunk(idx_v, out_v, n):
        @plsc.parallel_loop(0, n, 16 * GANG, unroll=UNROLL)
        def body(i):
            ivs = [idx_v[pl.ds(i + 16 * u, 16)] for u in range(GANG)]
            vals = [plsc.load_gather(row_v, [iv]) for iv in ivs]
            for u in range(GANG):
                out_v[pl.ds(i + 16 * u, 16)] = vals[u]

    passes = []
    phases = []
    ph = 0
    for t in range(DIMS_PER_W):
        j = wid * DIMS_PER_W + t
        for tab, idx_hbm, chs, out_off in (
                (in_tab_t, in_idx_hbm, CH_A, 0),
                (out_tab_t, bc_idx_hbm, CH_BC, BATCH)):
            passes.append((j, tab, idx_hbm, chs, out_off))
            phases.append(ph)
            ph = (ph + len(chs)) % 2

    def idx_copy(p, c):
        _, _, idx_hbm, chs, _ = passes[p]
        b = (phases[p] + c) % 2
        off = sum(chs[:c])
        return pltpu.async_copy(
            idx_hbm.at[pl.ds(off, chs[c])],
            idx_vs[b].at[pl.ds(0, chs[c])], isems[b])

    nxt_first = idx_copy(0, 0)
    for p, (j, tab, idx_hbm, chs, out_off) in enumerate(passes):
        pltpu.sync_copy(tab.at[j], row_v)
        nch = len(chs)
        ids = {phases[p] % 2: nxt_first}
        wds = [None] * nch
        nxt_first = None
        base = 0
        for c in range(nch):
            b = (phases[p] + c) % 2
            if c + 1 < nch:
                ids[(phases[p] + c + 1) % 2] = idx_copy(p, c + 1)
            ids[b].wait()
            if c - 2 >= 0:
                wds[c - 2].wait()
            gather_chunk(idx_vs[b], out_vs[b], chs[c])
            if c == nch - 1 and p + 1 < len(passes):
                nxt_first = idx_copy(p + 1, 0)
            wds[c] = pltpu.async_copy(
                out_vs[b].at[pl.ds(0, chs[c])],
                out_t.at[j, pl.ds(out_off + base, chs[c])], wsems[b])
            base += chs[c]
        for c in range(max(0, nch - 2), nch):
            wds[c].wait()


def kernel(input_words, output_words, noise_words, in_embed_weight,
           out_embed_weight):
    bc_idx = jnp.concatenate(
        [output_words.astype(jnp.int32),
         noise_words.reshape(-1).astype(jnp.int32)], axis=0)
    mesh = plsc.VectorSubcoreMesh(core_axis_name="c", subcore_axis_name="s")
    f = pl.kernel(
        _gather_body,
        mesh=mesh,
        out_type=jax.ShapeDtypeStruct((N_EMBED, TOTAL), jnp.float32),
        scratch_types=[
            pltpu.VMEM((N_VOCAB,), jnp.float32),
            [pltpu.VMEM((IC,), jnp.int32)] * 2,
            [pltpu.VMEM((IC,), jnp.float32)] * 2,
            [pltpu.SemaphoreType.DMA] * 2,
            [pltpu.SemaphoreType.DMA] * 2,
        ],
        compiler_params=pltpu.CompilerParams(use_tc_tiling_on_sc=True,
                                             needs_layout_passes=False),
    )
    out_t = f(
        input_words.astype(jnp.int32),
        bc_idx,
        in_embed_weight.T,
        out_embed_weight.T,
    )
    return out_t.T


# speedup vs baseline: 1.0900x; 1.0301x over previous
import jax
import jax.numpy as jnp
from jax import lax
from jax.experimental import pallas as pl
from jax.experimental.pallas import tpu as pltpu
from jax.experimental.pallas import tpu_sc as plsc

N_VOCAB = 100000
N_EMBED = 64
BATCH = 16384
N_SAMPLES = 5

NC = 2
NS = 16
NW = NC * NS
DIMS_PER_W = N_EMBED // NW

TOTAL = BATCH * (2 + N_SAMPLES)
N_BC = BATCH * (1 + N_SAMPLES)

IC = 7680
CH_A = (7680, 7680, 1024)
CH_BC = (7680,) * 12 + (6144,)
UNROLL = 2
GANG = 4


def _gather_body(in_idx_hbm, bc_idx_hbm, in_tab_t, out_tab_t, out_t,
                 row_v, idx_vs, out_vs, isems, wsems, rsem):
    wid = lax.axis_index("s") * NC + lax.axis_index("c")

    def gather_chunk(idx_v, out_v, n):
        @plsc.parallel_loop(0, n, 16 * GANG, unroll=UNROLL)
        def body(i):
            ivs = [idx_v[pl.ds(i + 16 * u, 16)] for u in range(GANG)]
            vals = [plsc.load_gather(row_v, [iv]) for iv in ivs]
            for u in range(GANG):
                out_v[pl.ds(i + 16 * u, 16)] = vals[u]

    passes = []
    phases = []
    ph = 0
    for t in range(DIMS_PER_W):
        j = wid * DIMS_PER_W + t
        for tab, idx_hbm, chs, out_off in (
                (in_tab_t, in_idx_hbm, CH_A, 0),
                (out_tab_t, bc_idx_hbm, CH_BC, BATCH)):
            passes.append((j, tab, idx_hbm, chs, out_off))
            phases.append(ph)
            ph = (ph + len(chs)) % 2

    def idx_copy(p, c):
        _, _, idx_hbm, chs, _ = passes[p]
        b = (phases[p] + c) % 2
        off = sum(chs[:c])
        return pltpu.async_copy(
            idx_hbm.at[pl.ds(off, chs[c])],
            idx_vs[b].at[pl.ds(0, chs[c])], isems[b])

    def row_copy(p):
        j, tab, _, _, _ = passes[p]
        return pltpu.async_copy(tab.at[j], row_v, rsem)

    nxt_first = idx_copy(0, 0)
    nxt_row = row_copy(0)
    for p, (j, tab, idx_hbm, chs, out_off) in enumerate(passes):
        nxt_row.wait()
        nch = len(chs)
        ids = {phases[p] % 2: nxt_first}
        wds = [None] * nch
        nxt_first = None
        base = 0
        for c in range(nch):
            b = (phases[p] + c) % 2
            if c + 1 < nch:
                ids[(phases[p] + c + 1) % 2] = idx_copy(p, c + 1)
            ids[b].wait()
            if c - 2 >= 0:
                wds[c - 2].wait()
            gather_chunk(idx_vs[b], out_vs[b], chs[c])
            if c == nch - 1 and p + 1 < len(passes):
                nxt_first = idx_copy(p + 1, 0)
                nxt_row = row_copy(p + 1)
            wds[c] = pltpu.async_copy(
                out_vs[b].at[pl.ds(0, chs[c])],
                out_t.at[j, pl.ds(out_off + base, chs[c])], wsems[b])
            base += chs[c]
        for c in range(max(0, nch - 2), nch):
            wds[c].wait()


def kernel(input_words, output_words, noise_words, in_embed_weight,
           out_embed_weight):
    bc_idx = jnp.concatenate(
        [output_words.astype(jnp.int32),
         noise_words.reshape(-1).astype(jnp.int32)], axis=0)
    mesh = plsc.VectorSubcoreMesh(core_axis_name="c", subcore_axis_name="s")
    f = pl.kernel(
        _gather_body,
        mesh=mesh,
        out_type=jax.ShapeDtypeStruct((N_EMBED, TOTAL), jnp.float32),
        scratch_types=[
            pltpu.VMEM((N_VOCAB,), jnp.float32),
            [pltpu.VMEM((IC,), jnp.int32)] * 2,
            [pltpu.VMEM((IC,), jnp.float32)] * 2,
            [pltpu.SemaphoreType.DMA] * 2,
            [pltpu.SemaphoreType.DMA] * 2,
            pltpu.SemaphoreType.DMA,
        ],
        compiler_params=pltpu.CompilerParams(use_tc_tiling_on_sc=True,
                                             needs_layout_passes=False),
    )
    out_t = f(
        input_words.astype(jnp.int32),
        bc_idx,
        in_embed_weight.T,
        out_embed_weight.T,
    )
    return out_t.T
